# baseline (speedup 1.0000x reference)
import functools

import jax
import jax.numpy as jnp
from jax import lax
from jax.experimental import pallas as pl
from jax.experimental.pallas import tpu as pltpu
from jax.experimental.pallas import tpu_sc as plsc

B = 16384
F = 26
D = 32

_INFO = plsc.get_sparse_core_info()
NC = _INFO.num_cores
NS = _INFO.num_subcores
NW = NC * NS
B_PER_W = B // NW
NB = 4
IDXLEN = NB * F
NGRP = B_PER_W // NB
NBUF = 4
NOUTER = NGRP // NBUF

assert B_PER_W * NW == B and NGRP * NB == B_PER_W and NOUTER * NBUF == NGRP

_mesh = plsc.VectorSubcoreMesh(core_axis_name="c", subcore_axis_name="s")


@functools.partial(
    pl.kernel,
    mesh=_mesh,
    out_type=jax.ShapeDtypeStruct((B, F, D), jnp.float32),
    compiler_params=pltpu.CompilerParams(use_tc_tiling_on_sc=False),
    scratch_types=[
        pltpu.VMEM((NGRP, IDXLEN), jnp.int32),
        pltpu.VMEM((NBUF, IDXLEN, D), jnp.float32),
        pltpu.SemaphoreType.DMA((NBUF,)),
        pltpu.SemaphoreType.DMA((NBUF,)),
    ],
)
def _gather_kernel(x_hbm, table_hbm, out_hbm, idx_v, rows_v, gsem, wsem):
    wid = lax.axis_index("s") * NC + lax.axis_index("c")
    b_base = wid * B_PER_W

    pltpu.sync_copy(x_hbm.at[wid], idx_v)

    def start_gather(g, buf):
        pltpu.make_async_copy(
            table_hbm.at[idx_v.at[g]], rows_v.at[buf], gsem.at[buf]
        ).start()

    def wait_gather(g, buf):
        pltpu.make_async_copy(
            table_hbm.at[idx_v.at[g]], rows_v.at[buf], gsem.at[buf]
        ).wait()

    def write_desc(g, buf, i):
        return pltpu.make_async_copy(
            rows_v.at[buf].at[pl.ds(i * F, F)],
            out_hbm.at[b_base + g * NB + i],
            wsem.at[buf],
        )

    for buf in range(NBUF):
        start_gather(buf, buf)

    def outer(o, _):
        for buf in range(NBUF):
            g = o * NBUF + buf
            wait_gather(g, buf)
            for i in range(NB):
                write_desc(g, buf, i).start()
        for buf in range(NBUF):
            g = o * NBUF + buf
            ng = g + NBUF

            @pl.when(ng < NGRP)
            def _():
                for i in range(NB):
                    write_desc(g, buf, i).wait()
                start_gather(ng, buf)
        return _

    lax.fori_loop(0, NOUTER, outer, None)

    for buf in range(NBUF):
        g = NGRP - NBUF + buf
        for i in range(NB):
            write_desc(g, buf, i).wait()


def kernel(x, table):
    slab = x.reshape(NW, NGRP, IDXLEN)
    return _gather_kernel(slab, table)


# speedup vs baseline: 1.0118x; 1.0118x over previous
import functools

import jax
import jax.numpy as jnp
from jax import lax
from jax.experimental import pallas as pl
from jax.experimental.pallas import tpu as pltpu
from jax.experimental.pallas import tpu_sc as plsc

B = 16384
F = 26
D = 32
N = B * F

_INFO = plsc.get_sparse_core_info()
NC = _INFO.num_cores
NS = _INFO.num_subcores
NW = NC * NS
PER_W = N // NW
CHUNK = 128
NCHUNK = PER_W // CHUNK
NBUF = 8
NGROUP = NCHUNK // NBUF

assert PER_W * NW == N and NCHUNK * CHUNK == PER_W and NGROUP * NBUF == NCHUNK

_mesh = plsc.VectorSubcoreMesh(core_axis_name="c", subcore_axis_name="s")


@functools.partial(
    pl.kernel,
    mesh=_mesh,
    out_type=jax.ShapeDtypeStruct((N, D), jnp.float32),
    compiler_params=pltpu.CompilerParams(use_tc_tiling_on_sc=False),
    scratch_types=[
        pltpu.VMEM((NCHUNK, CHUNK), jnp.int32),
        pltpu.VMEM((NBUF, CHUNK, D), jnp.float32),
        pltpu.SemaphoreType.DMA((NBUF,)),
        pltpu.SemaphoreType.DMA((NBUF,)),
    ],
)
def _gather_kernel(x_hbm, table_hbm, out_hbm, idx_v, rows_v, gsem, wsem):
    wid = lax.axis_index("s") * NC + lax.axis_index("c")
    base = wid * PER_W

    pltpu.sync_copy(x_hbm.at[pl.ds(wid * NCHUNK, NCHUNK)], idx_v)

    def start_gather(j, b):
        pltpu.make_async_copy(
            table_hbm.at[idx_v.at[j]], rows_v.at[b], gsem.at[b]
        ).start()

    def wait_gather(j, b):
        pltpu.make_async_copy(
            table_hbm.at[idx_v.at[j]], rows_v.at[b], gsem.at[b]
        ).wait()

    def start_write(j, b):
        pltpu.make_async_copy(
            rows_v.at[b], out_hbm.at[pl.ds(base + j * CHUNK, CHUNK)], wsem.at[b]
        ).start()

    def wait_write(j, b):
        pltpu.make_async_copy(
            rows_v.at[b], out_hbm.at[pl.ds(base + j * CHUNK, CHUNK)], wsem.at[b]
        ).wait()

    for b in range(NBUF):
        start_gather(b, b)

    def group(g, _):
        for b in range(NBUF):
            j = g * NBUF + b
            wait_gather(j, b)
            start_write(j, b)
        for b in range(NBUF):
            j = g * NBUF + b
            nj = j + NBUF

            @pl.when(nj < NCHUNK)
            def _():
                wait_write(j, b)
                start_gather(nj, b)
        return _

    lax.fori_loop(0, NGROUP, group, None)

    for b in range(NBUF):
        wait_write(NCHUNK - NBUF + b, b)


def kernel(x, table):
    flat = x.reshape(NW * NCHUNK, CHUNK)
    out = _gather_kernel(flat, table)
    return out.reshape(B, F, D)
